# Initial kernel scaffold; baseline (speedup 1.0000x reference)
import jax
import jax.numpy as jnp
from jax.experimental import pallas as pl


def kernel(x, edge_index, gcn_W0, gcn_b0, gcn_W1, gcn_b1, gcn_W2, gcn_b2, red_W0, red_b0, red_W1, red_b1, gate_W0, gate_W1):
    raise NotImplementedError("write your pallas kernel here")


# speedup vs baseline: 30.8793x; 30.8793x over previous
import functools

import jax
import jax.numpy as jnp
from jax import lax
from jax.experimental import pallas as pl
from jax.experimental.pallas import tpu as pltpu
from jax.experimental.pallas import tpu_sc as plsc

N = 10000
E = 320000
DIN = 128
DH = 16
DOUT = 64

NC, NS, L = 2, 16, 16
NW = NC * NS
NP = 10240
ROWS_S = NP // NS
CK = 125
ECH = E // CK
WCH = ECH // NW

_f32 = jnp.float32
_mesh = plsc.VectorSubcoreMesh(core_axis_name="c", subcore_axis_name="s",
                               num_cores=NC, num_subcores=NS)
_sc_params = pltpu.CompilerParams(use_tc_tiling_on_sc=False)


def _zero_acc_slice(stage_v, acc_sh, sid):
    def zrow(i, c):
        stage_v[i, :] = jnp.zeros((L,), _f32)
        return c
    lax.fori_loop(0, ROWS_S, zrow, 0)
    pltpu.sync_copy(stage_v, acc_sh.at[pl.ds(sid * ROWS_S, ROWS_S)])


@functools.partial(
    pl.kernel,
    out_type=jax.ShapeDtypeStruct((NC * NP, L), _f32),
    mesh=_mesh,
    compiler_params=_sc_params,
    scratch_types=[
        pltpu.VMEM((WCH, CK), jnp.int32),
        pltpu.VMEM((CK, L), _f32),
        pltpu.VMEM((ROWS_S, L), _f32),
        pltpu.VMEM_SHARED((NP, L), _f32),
    ],
)
def _deg(dst_hbm, out_hbm, dst_v, ones_v, stage_v, acc_sh):
    cid = lax.axis_index("c")
    sid = lax.axis_index("s")
    wid = cid * NS + sid

    _zero_acc_slice(stage_v, acc_sh, sid)

    def orow(i, c):
        ones_v[i, :] = jnp.ones((L,), _f32)
        return c
    lax.fori_loop(0, CK, orow, 0)
    pltpu.sync_copy(dst_hbm.at[pl.ds(wid * WCH, WCH)], dst_v)
    plsc.subcore_barrier()

    def step(j, c):
        pltpu.sync_copy(ones_v, acc_sh.at[dst_v.at[j]], add=True)
        return c
    lax.fori_loop(0, WCH, step, 0)

    plsc.subcore_barrier()
    sl = pl.ds(sid * ROWS_S, ROWS_S)
    pltpu.sync_copy(acc_sh.at[sl],
                    out_hbm.at[pl.ds(cid * NP + sid * ROWS_S, ROWS_S)])


@functools.partial(
    pl.kernel,
    out_type=jax.ShapeDtypeStruct((NC * NP, L), _f32),
    mesh=_mesh,
    compiler_params=_sc_params,
    scratch_types=[
        pltpu.VMEM((WCH, CK), jnp.int32),
        pltpu.VMEM((WCH, CK), jnp.int32),
        pltpu.VMEM((CK, L), _f32),
        pltpu.VMEM((ROWS_S, L), _f32),
        pltpu.VMEM_SHARED((NP, L), _f32),
        pltpu.SemaphoreType.DMA,
    ],
)
def _agg(tab_hbm, src_hbm, dst_hbm, out_hbm,
         src_v, dst_v, rows_v, stage_v, acc_sh, gsem):
    cid = lax.axis_index("c")
    sid = lax.axis_index("s")
    wid = cid * NS + sid
    sl = pl.ds(sid * ROWS_S, ROWS_S)

    _zero_acc_slice(stage_v, acc_sh, sid)
    pltpu.sync_copy(src_hbm.at[pl.ds(wid * WCH, WCH)], src_v)
    pltpu.sync_copy(dst_hbm.at[pl.ds(wid * WCH, WCH)], dst_v)
    plsc.subcore_barrier()

    def step(j, c):
        pltpu.async_copy(tab_hbm.at[src_v.at[j]], rows_v, gsem).wait()
        pltpu.sync_copy(rows_v, acc_sh.at[dst_v.at[j]], add=True)
        return c
    lax.fori_loop(0, WCH, step, 0)

    plsc.subcore_barrier()
    pltpu.sync_copy(acc_sh.at[sl],
                    out_hbm.at[pl.ds(cid * NP + sid * ROWS_S, ROWS_S)])


def _prep_body(x_ref, w0_ref, wr0_ref, wr1_ref, rb0_ref, rb1_ref, degacc_ref,
               v0_ref, r0_ref, r1_ref, d_ref):
    x = x_ref[...]
    deg = degacc_ref[:NP, :] + degacc_ref[NP:, :] + 1.0
    dinv = lax.rsqrt(deg)
    d_ref[...] = dinv
    v0_ref[...] = dinv * jnp.dot(x, w0_ref[...], preferred_element_type=_f32)
    r0_ref[...] = jnp.dot(x, wr0_ref[...], preferred_element_type=_f32) + rb0_ref[...]
    r1_ref[...] = jnp.dot(x, wr1_ref[...], preferred_element_type=_f32) + rb1_ref[...]


def _mid_body(inv_wnorm, acc_ref, v_ref, d_ref, h0_ref, b_ref, gw_ref, w1_ref,
              vout_ref):
    d = d_ref[...]
    h1 = jnp.maximum(
        d * (acc_ref[:NP, :] + acc_ref[NP:, :] + v_ref[...]) + b_ref[...], 0.0)
    h0 = h0_ref[...]
    gw = gw_ref[...]
    if inv_wnorm:
        scale = lax.rsqrt(jnp.sum(gw * gw))
    else:
        scale = 1.0
    z = (jnp.dot(h1, gw[:DH, :], preferred_element_type=_f32)
         + jnp.dot(h0, gw[DH:, :], preferred_element_type=_f32)) * scale
    g = jax.nn.sigmoid(z)
    h = g * h1 + (1.0 - g) * h0
    vout_ref[...] = d * jnp.dot(h, w1_ref[...], preferred_element_type=_f32)


def _mid1_body(acc_ref, v_ref, d_ref, h0_ref, b_ref, gw_ref, vout_ref):
    d = d_ref[...]
    h1 = jnp.maximum(
        d * (acc_ref[:NP, :] + acc_ref[NP:, :] + v_ref[...]) + b_ref[...], 0.0)
    h0 = h0_ref[...]
    gw = gw_ref[...]
    scale = lax.rsqrt(jnp.sum(gw * gw))
    z = (jnp.dot(h1, gw[:DH, :], preferred_element_type=_f32)
         + jnp.dot(h0, gw[DH:, :], preferred_element_type=_f32)) * scale
    g = jax.nn.sigmoid(z)
    h = g * h1 + (1.0 - g) * h0
    vout_ref[...] = d * h


def _final_body(acc_ref, v_ref, d_ref, w2_ref, b2_ref, out_ref):
    a2 = d_ref[...] * (acc_ref[:NP, :] + acc_ref[NP:, :] + v_ref[...])
    o = jnp.dot(a2, w2_ref[...], preferred_element_type=_f32) + b2_ref[...]
    m = jnp.max(o, axis=1, keepdims=True)
    e = o - m
    out_ref[...] = e - jnp.log(jnp.sum(jnp.exp(e), axis=1, keepdims=True))


def _np16(n=1):
    return jax.ShapeDtypeStruct((NP, L), _f32)


def kernel(x, edge_index, gcn_W0, gcn_b0, gcn_W1, gcn_b1, gcn_W2, gcn_b2,
           red_W0, red_b0, red_W1, red_b1, gate_W0, gate_W1):
    x = x.astype(_f32)
    ei = edge_index.astype(jnp.int32)
    src2 = ei[0].reshape(ECH, CK)
    dst2 = ei[1].reshape(ECH, CK)
    xp = jnp.pad(x, ((0, NP - N), (0, 0)))

    degacc = _deg(dst2)

    v0, r0, r1, d = pl.pallas_call(
        _prep_body,
        out_shape=(_np16(), _np16(), _np16(), _np16()),
    )(xp, gcn_W0, red_W0, red_W1, red_b0.reshape(1, DH), red_b1.reshape(1, DH),
      degacc)

    acc0 = _agg(v0, src2, dst2)
    v1 = pl.pallas_call(
        functools.partial(_mid_body, False),
        out_shape=_np16(),
    )(acc0, v0, d, r0, gcn_b0.reshape(1, DH), gate_W0, gcn_W1)

    acc1 = _agg(v1, src2, dst2)
    v2 = pl.pallas_call(
        _mid1_body,
        out_shape=_np16(),
    )(acc1, v1, d, r1, gcn_b1.reshape(1, DH), gate_W1)

    acc2 = _agg(v2, src2, dst2)
    out = pl.pallas_call(
        _final_body,
        out_shape=jax.ShapeDtypeStruct((NP, DOUT), _f32),
    )(acc2, v2, d, gcn_W2, gcn_b2.reshape(1, DOUT))

    return out[:N]


# speedup vs baseline: 51.4739x; 1.6669x over previous
import functools

import jax
import jax.numpy as jnp
from jax import lax
from jax.experimental import pallas as pl
from jax.experimental.pallas import tpu as pltpu
from jax.experimental.pallas import tpu_sc as plsc

N = 10000
E = 320000
DIN = 128
DH = 16
DOUT = 64

NC, NS, L = 2, 16, 16
NW = NC * NS
NP = 10240
ROWS_S = NP // NS
CK = 125
ECH = E // CK
WCH = ECH // NW

_f32 = jnp.float32
_mesh = plsc.VectorSubcoreMesh(core_axis_name="c", subcore_axis_name="s",
                               num_cores=NC, num_subcores=NS)
_sc_params = pltpu.CompilerParams(use_tc_tiling_on_sc=False)


def _zero_acc_slice(stage_v, acc_sh, sid):
    def zrow(i, c):
        stage_v[i, :] = jnp.zeros((L,), _f32)
        return c
    lax.fori_loop(0, ROWS_S, zrow, 0)
    pltpu.sync_copy(stage_v, acc_sh.at[pl.ds(sid * ROWS_S, ROWS_S)])


@functools.partial(
    pl.kernel,
    out_type=jax.ShapeDtypeStruct((NC * NP, L), _f32),
    mesh=_mesh,
    compiler_params=_sc_params,
    scratch_types=[
        pltpu.VMEM((WCH, CK), jnp.int32),
        pltpu.VMEM((CK, L), _f32),
        pltpu.VMEM((ROWS_S, L), _f32),
        pltpu.VMEM_SHARED((NP, L), _f32),
    ],
)
def _deg(dst_hbm, out_hbm, dst_v, ones_v, stage_v, acc_sh):
    cid = lax.axis_index("c")
    sid = lax.axis_index("s")
    wid = cid * NS + sid

    _zero_acc_slice(stage_v, acc_sh, sid)

    def orow(i, c):
        ones_v[i, :] = jnp.ones((L,), _f32)
        return c
    lax.fori_loop(0, CK, orow, 0)
    pltpu.sync_copy(dst_hbm.at[pl.ds(wid * WCH, WCH)], dst_v)
    plsc.subcore_barrier()

    def step(j, c):
        pltpu.sync_copy(ones_v, acc_sh.at[dst_v.at[j]], add=True)
        return c
    lax.fori_loop(0, WCH, step, 0)

    plsc.subcore_barrier()
    sl = pl.ds(sid * ROWS_S, ROWS_S)
    pltpu.sync_copy(acc_sh.at[sl],
                    out_hbm.at[pl.ds(cid * NP + sid * ROWS_S, ROWS_S)])


@functools.partial(
    pl.kernel,
    out_type=jax.ShapeDtypeStruct((NC * NP, L), _f32),
    mesh=_mesh,
    compiler_params=_sc_params,
    scratch_types=[
        pltpu.VMEM((WCH, CK), jnp.int32),
        pltpu.VMEM((WCH, CK), jnp.int32),
        pltpu.VMEM((2, CK, L), _f32),
        pltpu.VMEM((ROWS_S, L), _f32),
        pltpu.VMEM_SHARED((NP, L), _f32),
        pltpu.VMEM_SHARED((NP, L), _f32),
        pltpu.SemaphoreType.DMA,
    ],
)
def _agg(tab_hbm, src_hbm, dst_hbm, out_hbm,
         src_v, dst_v, rows_v, stage_v, acc_sh, tab_sh, gsem):
    cid = lax.axis_index("c")
    sid = lax.axis_index("s")
    wid = cid * NS + sid
    sl = pl.ds(sid * ROWS_S, ROWS_S)

    _zero_acc_slice(stage_v, acc_sh, sid)
    pltpu.sync_copy(tab_hbm.at[sl], tab_sh.at[sl])
    pltpu.sync_copy(src_hbm.at[pl.ds(wid * WCH, WCH)], src_v)
    pltpu.sync_copy(dst_hbm.at[pl.ds(wid * WCH, WCH)], dst_v)
    plsc.subcore_barrier()

    pltpu.async_copy(tab_sh.at[src_v.at[0]], rows_v.at[0], gsem)

    def step(j, c):
        b = lax.rem(j, 2)
        nb = lax.rem(j + 1, 2)

        @pl.when(j + 1 < WCH)
        def _():
            pltpu.async_copy(tab_sh.at[src_v.at[j + 1]], rows_v.at[nb], gsem)

        pltpu.make_async_copy(tab_sh.at[src_v.at[j]], rows_v.at[b], gsem).wait()
        pltpu.sync_copy(rows_v.at[b], acc_sh.at[dst_v.at[j]], add=True)
        return c
    lax.fori_loop(0, WCH, step, 0)

    plsc.subcore_barrier()
    pltpu.sync_copy(acc_sh.at[sl],
                    out_hbm.at[pl.ds(cid * NP + sid * ROWS_S, ROWS_S)])


def _prep_body(x_ref, w0_ref, wr0_ref, wr1_ref, rb0_ref, rb1_ref, degacc_ref,
               v0_ref, r0_ref, r1_ref, d_ref):
    x = x_ref[...]
    deg = degacc_ref[:NP, :] + degacc_ref[NP:, :] + 1.0
    dinv = lax.rsqrt(deg)
    d_ref[...] = dinv
    v0_ref[...] = dinv * jnp.dot(x, w0_ref[...], preferred_element_type=_f32)
    r0_ref[...] = jnp.dot(x, wr0_ref[...], preferred_element_type=_f32) + rb0_ref[...]
    r1_ref[...] = jnp.dot(x, wr1_ref[...], preferred_element_type=_f32) + rb1_ref[...]


def _mid_body(inv_wnorm, acc_ref, v_ref, d_ref, h0_ref, b_ref, gw_ref, w1_ref,
              vout_ref):
    d = d_ref[...]
    h1 = jnp.maximum(
        d * (acc_ref[:NP, :] + acc_ref[NP:, :] + v_ref[...]) + b_ref[...], 0.0)
    h0 = h0_ref[...]
    gw = gw_ref[...]
    if inv_wnorm:
        scale = lax.rsqrt(jnp.sum(gw * gw))
    else:
        scale = 1.0
    z = (jnp.dot(h1, gw[:DH, :], preferred_element_type=_f32)
         + jnp.dot(h0, gw[DH:, :], preferred_element_type=_f32)) * scale
    g = jax.nn.sigmoid(z)
    h = g * h1 + (1.0 - g) * h0
    vout_ref[...] = d * jnp.dot(h, w1_ref[...], preferred_element_type=_f32)


def _mid1_body(acc_ref, v_ref, d_ref, h0_ref, b_ref, gw_ref, vout_ref):
    d = d_ref[...]
    h1 = jnp.maximum(
        d * (acc_ref[:NP, :] + acc_ref[NP:, :] + v_ref[...]) + b_ref[...], 0.0)
    h0 = h0_ref[...]
    gw = gw_ref[...]
    scale = lax.rsqrt(jnp.sum(gw * gw))
    z = (jnp.dot(h1, gw[:DH, :], preferred_element_type=_f32)
         + jnp.dot(h0, gw[DH:, :], preferred_element_type=_f32)) * scale
    g = jax.nn.sigmoid(z)
    h = g * h1 + (1.0 - g) * h0
    vout_ref[...] = d * h


def _final_body(acc_ref, v_ref, d_ref, w2_ref, b2_ref, out_ref):
    a2 = d_ref[...] * (acc_ref[:NP, :] + acc_ref[NP:, :] + v_ref[...])
    o = jnp.dot(a2, w2_ref[...], preferred_element_type=_f32) + b2_ref[...]
    m = jnp.max(o, axis=1, keepdims=True)
    e = o - m
    out_ref[...] = e - jnp.log(jnp.sum(jnp.exp(e), axis=1, keepdims=True))


def _np16(n=1):
    return jax.ShapeDtypeStruct((NP, L), _f32)


def kernel(x, edge_index, gcn_W0, gcn_b0, gcn_W1, gcn_b1, gcn_W2, gcn_b2,
           red_W0, red_b0, red_W1, red_b1, gate_W0, gate_W1):
    x = x.astype(_f32)
    ei = edge_index.astype(jnp.int32)
    src2 = ei[0].reshape(ECH, CK)
    dst2 = ei[1].reshape(ECH, CK)
    xp = jnp.pad(x, ((0, NP - N), (0, 0)))

    degacc = _deg(dst2)

    v0, r0, r1, d = pl.pallas_call(
        _prep_body,
        out_shape=(_np16(), _np16(), _np16(), _np16()),
    )(xp, gcn_W0, red_W0, red_W1, red_b0.reshape(1, DH), red_b1.reshape(1, DH),
      degacc)

    acc0 = _agg(v0, src2, dst2)
    v1 = pl.pallas_call(
        functools.partial(_mid_body, False),
        out_shape=_np16(),
    )(acc0, v0, d, r0, gcn_b0.reshape(1, DH), gate_W0, gcn_W1)

    acc1 = _agg(v1, src2, dst2)
    v2 = pl.pallas_call(
        _mid1_body,
        out_shape=_np16(),
    )(acc1, v1, d, r1, gcn_b1.reshape(1, DH), gate_W1)

    acc2 = _agg(v2, src2, dst2)
    out = pl.pallas_call(
        _final_body,
        out_shape=jax.ShapeDtypeStruct((NP, DOUT), _f32),
    )(acc2, v2, d, gcn_W2, gcn_b2.reshape(1, DOUT))

    return out[:N]


# speedup vs baseline: 69.5829x; 1.3518x over previous
import functools

import jax
import jax.numpy as jnp
from jax import lax
from jax.experimental import pallas as pl
from jax.experimental.pallas import tpu as pltpu
from jax.experimental.pallas import tpu_sc as plsc

N = 10000
E = 320000
DIN = 128
DH = 16
DOUT = 64

NC, NS, L = 2, 16, 16
NW = NC * NS
NP = 10240
RP = NP // 8
ROWS_S = NP // NS
CK = 125
ECH = E // CK
WCH = ECH // NW
GRID = 8
RB = RP // GRID

_f32 = jnp.float32
_mesh = plsc.VectorSubcoreMesh(core_axis_name="c", subcore_axis_name="s",
                               num_cores=NC, num_subcores=NS)
_sc_params = pltpu.CompilerParams(use_tc_tiling_on_sc=False)


def _zero_acc_slice(stage_v, acc_sh, sid):
    def zrow(i, c):
        stage_v[i, :] = jnp.zeros((L,), _f32)
        return c
    lax.fori_loop(0, ROWS_S, zrow, 0)
    pltpu.sync_copy(stage_v, acc_sh.at[pl.ds(sid * ROWS_S, ROWS_S)])


def _writeout(acc_sh, out0_hbm, out1_hbm, cid, sid):
    sl = pl.ds(sid * ROWS_S, ROWS_S)

    @pl.when(cid == 0)
    def _():
        pltpu.sync_copy(acc_sh.at[sl], out0_hbm.at[sl])

    @pl.when(cid == 1)
    def _():
        pltpu.sync_copy(acc_sh.at[sl], out1_hbm.at[sl])


@functools.partial(
    pl.kernel,
    out_type=(jax.ShapeDtypeStruct((NP, L), _f32),
              jax.ShapeDtypeStruct((NP, L), _f32)),
    mesh=_mesh,
    compiler_params=_sc_params,
    scratch_types=[
        pltpu.VMEM((WCH, CK), jnp.int32),
        pltpu.VMEM((CK, L), _f32),
        pltpu.VMEM((ROWS_S, L), _f32),
        pltpu.VMEM_SHARED((NP, L), _f32),
    ],
)
def _deg(dst_hbm, out0_hbm, out1_hbm, dst_v, ones_v, stage_v, acc_sh):
    cid = lax.axis_index("c")
    sid = lax.axis_index("s")
    wid = cid * NS + sid

    _zero_acc_slice(stage_v, acc_sh, sid)

    def orow(i, c):
        ones_v[i, :] = jnp.ones((L,), _f32)
        return c
    lax.fori_loop(0, CK, orow, 0)
    pltpu.sync_copy(dst_hbm.at[pl.ds(wid * WCH, WCH)], dst_v)
    plsc.subcore_barrier()

    def step(j, c):
        pltpu.sync_copy(ones_v, acc_sh.at[dst_v.at[j]], add=True)
        return c
    lax.fori_loop(0, WCH, step, 0)

    plsc.subcore_barrier()
    _writeout(acc_sh, out0_hbm, out1_hbm, cid, sid)


@functools.partial(
    pl.kernel,
    out_type=(jax.ShapeDtypeStruct((NP, L), _f32),
              jax.ShapeDtypeStruct((NP, L), _f32)),
    mesh=_mesh,
    compiler_params=_sc_params,
    scratch_types=[
        pltpu.VMEM((WCH, CK), jnp.int32),
        pltpu.VMEM((WCH, CK), jnp.int32),
        pltpu.VMEM((2, CK, L), _f32),
        pltpu.VMEM((ROWS_S, L), _f32),
        pltpu.VMEM_SHARED((NP, L), _f32),
        pltpu.VMEM_SHARED((NP, L), _f32),
        pltpu.SemaphoreType.DMA,
    ],
)
def _agg(tab_hbm, src_hbm, dst_hbm, out0_hbm, out1_hbm,
         src_v, dst_v, rows_v, stage_v, acc_sh, tab_sh, gsem):
    cid = lax.axis_index("c")
    sid = lax.axis_index("s")
    wid = cid * NS + sid
    sl = pl.ds(sid * ROWS_S, ROWS_S)

    _zero_acc_slice(stage_v, acc_sh, sid)
    pltpu.sync_copy(tab_hbm.at[sl], tab_sh.at[sl])
    pltpu.sync_copy(src_hbm.at[pl.ds(wid * WCH, WCH)], src_v)
    pltpu.sync_copy(dst_hbm.at[pl.ds(wid * WCH, WCH)], dst_v)
    plsc.subcore_barrier()

    pltpu.async_copy(tab_sh.at[src_v.at[0]], rows_v.at[0], gsem)

    def step(j, c):
        b = lax.rem(j, 2)
        nb = lax.rem(j + 1, 2)

        @pl.when(j + 1 < WCH)
        def _():
            pltpu.async_copy(tab_sh.at[src_v.at[j + 1]], rows_v.at[nb], gsem)

        pltpu.make_async_copy(tab_sh.at[src_v.at[j]], rows_v.at[b], gsem).wait()
        pltpu.sync_copy(rows_v.at[b], acc_sh.at[dst_v.at[j]], add=True)
        return c
    lax.fori_loop(0, WCH, step, 0)

    plsc.subcore_barrier()
    _writeout(acc_sh, out0_hbm, out1_hbm, cid, sid)


def _prep_body(xk_ref, kall_ref, tb_ref, dg0_ref, dg1_ref,
               v0_ref, r0_ref, r1_ref, d_ref):
    deg = dg0_ref[...] + dg1_ref[...] + 1.0
    dinv = lax.rsqrt(deg)
    d_ref[...] = dinv
    p = jnp.dot(xk_ref[...], kall_ref[...], preferred_element_type=_f32)
    v0_ref[...] = dinv * p[:, :128]
    r0_ref[...] = p[:, 128:256] + tb_ref[0:1, :128]
    r1_ref[...] = p[:, 256:384] + tb_ref[0:1, 128:256]


def _mid_body(last, a0_ref, a1_ref, v_ref, d_ref, h0_ref, tb_ref, gw_ref,
              kga_ref, kgb_ref, kb8_ref, kw1_ref, vout_ref):
    d = d_ref[...]
    h1 = jnp.maximum(
        d * (a0_ref[...] + a1_ref[...] + v_ref[...]) + tb_ref[0:1, :], 0.0)
    h0 = h0_ref[...]
    if last:
        gw = gw_ref[...]
        scale = lax.rsqrt(jnp.sum(gw * gw))
    else:
        scale = 1.0
    zz = (jnp.dot(h1, kga_ref[...], preferred_element_type=_f32)
          + jnp.dot(h0, kgb_ref[...], preferred_element_type=_f32)) * scale
    g = jax.nn.sigmoid(zz)
    gp = jnp.dot(g, kb8_ref[...], preferred_element_type=_f32)
    h = gp * h1 + (1.0 - gp) * h0
    if last:
        vout_ref[...] = d * h
    else:
        vout_ref[...] = d * jnp.dot(h, kw1_ref[...], preferred_element_type=_f32)


def _final_body(a0_ref, a1_ref, v_ref, d_ref, kw2_ref, tb2_ref, out_ref):
    a2 = d_ref[...] * (a0_ref[...] + a1_ref[...] + v_ref[...])
    o = jnp.dot(a2, kw2_ref[...], preferred_element_type=_f32) + tb2_ref[0:1, :]
    o3 = o.reshape(o.shape[0], 8, DOUT)
    m = jnp.max(o3, axis=2, keepdims=True)
    e = o3 - m
    ls = jnp.log(jnp.sum(jnp.exp(e), axis=2, keepdims=True))
    out_ref[...] = (e - ls).reshape(o.shape[0], 8 * DOUT)


def _rows(i):
    return (i, 0)


def _whole(i):
    return (0, 0)


def _pk(shape=(RB, 128)):
    return pl.BlockSpec(shape, _rows)


def _w(shape):
    return pl.BlockSpec(shape, _whole)


def kernel(x, edge_index, gcn_W0, gcn_b0, gcn_W1, gcn_b1, gcn_W2, gcn_b2,
           red_W0, red_b0, red_W1, red_b1, gate_W0, gate_W1):
    x = x.astype(_f32)
    ei = edge_index.astype(jnp.int32)
    src2 = ei[0].reshape(ECH, CK)
    dst2 = ei[1].reshape(ECH, CK)
    xk = jnp.pad(x, ((0, NP - N), (0, 0))).reshape(RP, 8 * DIN)

    eye8 = jnp.eye(8, dtype=_f32)
    kall = jnp.concatenate(
        [jnp.kron(eye8, w) for w in (gcn_W0, red_W0, red_W1)], axis=1)
    tb = jnp.concatenate(
        [jnp.tile(b, 8) for b in (red_b0, red_b1)]).reshape(1, 256)
    kb8 = jnp.kron(eye8, jnp.ones((1, DH), _f32))

    dg0, dg1 = _deg(dst2)
    dg0p, dg1p = dg0.reshape(RP, 128), dg1.reshape(RP, 128)

    v0p, r0p, r1p, dp = pl.pallas_call(
        _prep_body,
        grid=(GRID,),
        in_specs=[_pk((RB, 8 * DIN)), _w((8 * DIN, 384)), _w((1, 256)),
                  _pk(), _pk()],
        out_specs=(_pk(), _pk(), _pk(), _pk()),
        out_shape=tuple(jax.ShapeDtypeStruct((RP, 128), _f32) for _ in range(4)),
    )(xk, kall, tb, dg0p, dg1p)

    def mid(last, a0, a1, vp, h0p, b, gw, w1):
        return pl.pallas_call(
            functools.partial(_mid_body, last),
            grid=(GRID,),
            in_specs=[_pk(), _pk(), _pk(), _pk(), _pk(), _w((1, 128)),
                      _w((2 * DH, 1)), _w((128, 8)), _w((128, 8)),
                      _w((8, 128)), _w((128, 128))],
            out_specs=_pk(),
            out_shape=jax.ShapeDtypeStruct((RP, 128), _f32),
        )(a0.reshape(RP, 128), a1.reshape(RP, 128), vp, dp, h0p,
          jnp.tile(b, 8).reshape(1, 128), gw,
          jnp.kron(eye8, gw[:DH, :]), jnp.kron(eye8, gw[DH:, :]), kb8,
          jnp.kron(eye8, w1))

    a0, a1 = _agg(v0p.reshape(NP, L), src2, dst2)
    v1p = mid(False, a0, a1, v0p, r0p, gcn_b0, gate_W0, gcn_W1)

    a0, a1 = _agg(v1p.reshape(NP, L), src2, dst2)
    v2p = mid(True, a0, a1, v1p, r1p, gcn_b1, gate_W1, gcn_W1)

    a0, a1 = _agg(v2p.reshape(NP, L), src2, dst2)
    outp = pl.pallas_call(
        _final_body,
        grid=(GRID,),
        in_specs=[_pk(), _pk(), _pk(), _pk(), _w((128, 8 * DOUT)),
                  _w((1, 8 * DOUT))],
        out_specs=pl.BlockSpec((RB, 8 * DOUT), _rows),
        out_shape=jax.ShapeDtypeStruct((RP, 8 * DOUT), _f32),
    )(a0.reshape(RP, 128), a1.reshape(RP, 128), v2p, dp,
      jnp.kron(eye8, gcn_W2), jnp.tile(gcn_b2, 8).reshape(1, 8 * DOUT))

    return outp.reshape(NP, DOUT)[:N]


# speedup vs baseline: 71.2648x; 1.0242x over previous
import functools

import jax
import jax.numpy as jnp
from jax import lax
from jax.experimental import pallas as pl
from jax.experimental.pallas import tpu as pltpu
from jax.experimental.pallas import tpu_sc as plsc

N = 10000
E = 320000
DIN = 128
DH = 16
DOUT = 64

NC, NS, L = 2, 16, 16
NW = NC * NS
NP = 10240
RP = NP // 8
ROWS_S = NP // NS
CK = 125
ECH = E // CK
WCH = ECH // NW
GRID = 8
RB = RP // GRID

_f32 = jnp.float32
_mesh = plsc.VectorSubcoreMesh(core_axis_name="c", subcore_axis_name="s",
                               num_cores=NC, num_subcores=NS)
_sc_params = pltpu.CompilerParams(use_tc_tiling_on_sc=False)


def _zero_acc_slice(stage_v, acc_sh, sid):
    def zrow(i, c):
        stage_v[i, :] = jnp.zeros((L,), _f32)
        return c
    lax.fori_loop(0, ROWS_S, zrow, 0)
    pltpu.sync_copy(stage_v, acc_sh.at[pl.ds(sid * ROWS_S, ROWS_S)])


def _writeout(acc_sh, out0_hbm, out1_hbm, cid, sid):
    sl = pl.ds(sid * ROWS_S, ROWS_S)

    @pl.when(cid == 0)
    def _():
        pltpu.sync_copy(acc_sh.at[sl], out0_hbm.at[sl])

    @pl.when(cid == 1)
    def _():
        pltpu.sync_copy(acc_sh.at[sl], out1_hbm.at[sl])


@functools.partial(
    pl.kernel,
    out_type=(jax.ShapeDtypeStruct((NP, L), _f32),
              jax.ShapeDtypeStruct((NP, L), _f32)),
    mesh=_mesh,
    compiler_params=_sc_params,
    scratch_types=[
        pltpu.VMEM((WCH, CK), jnp.int32),
        pltpu.VMEM((CK, L), _f32),
        pltpu.VMEM((ROWS_S, L), _f32),
        pltpu.VMEM_SHARED((NP, L), _f32),
    ],
)
def _deg(dst_hbm, out0_hbm, out1_hbm, dst_v, ones_v, stage_v, acc_sh):
    cid = lax.axis_index("c")
    sid = lax.axis_index("s")
    wid = cid * NS + sid

    _zero_acc_slice(stage_v, acc_sh, sid)

    def orow(i, c):
        ones_v[i, :] = jnp.ones((L,), _f32)
        return c
    lax.fori_loop(0, CK, orow, 0)
    pltpu.sync_copy(dst_hbm.at[pl.ds(wid * WCH, WCH)], dst_v)
    plsc.subcore_barrier()

    def step(j, c):
        pltpu.sync_copy(ones_v, acc_sh.at[dst_v.at[j]], add=True)
        return c
    lax.fori_loop(0, WCH, step, 0)

    plsc.subcore_barrier()
    _writeout(acc_sh, out0_hbm, out1_hbm, cid, sid)


@functools.partial(
    pl.kernel,
    out_type=(jax.ShapeDtypeStruct((NP, L), _f32),
              jax.ShapeDtypeStruct((NP, L), _f32)),
    mesh=_mesh,
    compiler_params=_sc_params,
    scratch_types=[
        pltpu.VMEM((WCH, CK), jnp.int32),
        pltpu.VMEM((WCH, CK), jnp.int32),
        pltpu.VMEM((2, CK, L), _f32),
        pltpu.VMEM((ROWS_S, L), _f32),
        pltpu.VMEM_SHARED((NP, L), _f32),
        pltpu.VMEM_SHARED((NP, L), _f32),
        pltpu.SemaphoreType.DMA,
    ],
)
def _agg(tab_hbm, src_hbm, dst_hbm, out0_hbm, out1_hbm,
         src_v, dst_v, rows_v, stage_v, acc_sh, tab_sh, gsem):
    cid = lax.axis_index("c")
    sid = lax.axis_index("s")
    wid = cid * NS + sid
    sl = pl.ds(sid * ROWS_S, ROWS_S)

    _zero_acc_slice(stage_v, acc_sh, sid)
    pltpu.sync_copy(tab_hbm.at[sl], tab_sh.at[sl])
    pltpu.sync_copy(src_hbm.at[pl.ds(wid * WCH, WCH)], src_v)
    pltpu.sync_copy(dst_hbm.at[pl.ds(wid * WCH, WCH)], dst_v)
    plsc.subcore_barrier()

    pltpu.async_copy(tab_sh.at[src_v.at[0]], rows_v.at[0], gsem)

    def step(j, c):
        b = lax.rem(j, 2)
        nb = lax.rem(j + 1, 2)

        @pl.when(j + 1 < WCH)
        def _():
            pltpu.async_copy(tab_sh.at[src_v.at[j + 1]], rows_v.at[nb], gsem)

        pltpu.make_async_copy(tab_sh.at[src_v.at[j]], rows_v.at[b], gsem).wait()
        pltpu.sync_copy(rows_v.at[b], acc_sh.at[dst_v.at[j]], add=True)
        return c
    lax.fori_loop(0, WCH, step, 0)

    plsc.subcore_barrier()
    _writeout(acc_sh, out0_hbm, out1_hbm, cid, sid)


def _prepa_body(xk_ref, w0_ref, wr0_ref, wr1_ref, p_ref, kall_v):
    @pl.when(pl.program_id(0) == 0)
    def _():
        kall_v[...] = jnp.zeros((8 * DIN, 384), _f32)
        for a in range(8):
            r = pl.ds(a * DIN, DIN)
            kall_v[r, pl.ds(a * DH, DH)] = w0_ref[...]
            kall_v[r, pl.ds(128 + a * DH, DH)] = wr0_ref[...]
            kall_v[r, pl.ds(256 + a * DH, DH)] = wr1_ref[...]

    p_ref[...] = jnp.dot(xk_ref[...], kall_v[...], preferred_element_type=_f32)


def _prepb_body(p_ref, b0_ref, b1_ref, dg0_ref, dg1_ref,
                v0_ref, r0_ref, r1_ref, d_ref):
    deg = dg0_ref[...] + dg1_ref[...] + 1.0
    dinv = lax.rsqrt(deg)
    d_ref[...] = dinv
    p = p_ref[...]
    tb0 = jnp.tile(b0_ref[...], (1, 8))
    tb1 = jnp.tile(b1_ref[...], (1, 8))
    v0_ref[...] = dinv * p[:, :128]
    r0_ref[...] = p[:, 128:256] + tb0
    r1_ref[...] = p[:, 256:384] + tb1


def _mid_body(last, a0_ref, a1_ref, v_ref, d_ref, h0_ref, b_ref, gwa_ref,
              gwb_ref, w1_ref, vout_ref, kga_v, kgb_v, kb8_v, kw1_v):
    @pl.when(pl.program_id(0) == 0)
    def _():
        kga_v[...] = jnp.zeros((128, 8), _f32)
        kgb_v[...] = jnp.zeros((128, 8), _f32)
        kb8_v[...] = jnp.zeros((8, 128), _f32)
        kw1_v[...] = jnp.zeros((128, 128), _f32)
        for a in range(8):
            r = pl.ds(a * DH, DH)
            kga_v[r, pl.ds(a, 1)] = gwa_ref[...]
            kgb_v[r, pl.ds(a, 1)] = gwb_ref[...]
            kb8_v[pl.ds(a, 1), r] = jnp.ones((1, DH), _f32)
            kw1_v[r, r] = w1_ref[...]

    d = d_ref[...]
    h1 = jnp.maximum(
        d * (a0_ref[...] + a1_ref[...] + v_ref[...])
        + jnp.tile(b_ref[...], (1, 8)), 0.0)
    h0 = h0_ref[...]
    if last:
        gwa, gwb = gwa_ref[...], gwb_ref[...]
        scale = lax.rsqrt(jnp.sum(gwa * gwa) + jnp.sum(gwb * gwb))
    else:
        scale = 1.0
    zz = (jnp.dot(h1, kga_v[...], preferred_element_type=_f32)
          + jnp.dot(h0, kgb_v[...], preferred_element_type=_f32)) * scale
    g = jax.nn.sigmoid(zz)
    gp = jnp.dot(g, kb8_v[...], preferred_element_type=_f32)
    h = gp * h1 + (1.0 - gp) * h0
    if last:
        vout_ref[...] = d * h
    else:
        vout_ref[...] = d * jnp.dot(h, kw1_v[...], preferred_element_type=_f32)


def _final_body(a0_ref, a1_ref, v_ref, d_ref, w2_ref, b2_ref, out_ref, kw2_v):
    @pl.when(pl.program_id(0) == 0)
    def _():
        kw2_v[...] = jnp.zeros((128, 8 * DOUT), _f32)
        for a in range(8):
            kw2_v[pl.ds(a * DH, DH), pl.ds(a * DOUT, DOUT)] = w2_ref[...]

    a2 = d_ref[...] * (a0_ref[...] + a1_ref[...] + v_ref[...])
    o = (jnp.dot(a2, kw2_v[...], preferred_element_type=_f32)
         + jnp.tile(b2_ref[...], (1, 8)))
    o3 = o.reshape(o.shape[0], 8, DOUT)
    m = jnp.max(o3, axis=2, keepdims=True)
    e = o3 - m
    ls = jnp.log(jnp.sum(jnp.exp(e), axis=2, keepdims=True))
    out_ref[...] = (e - ls).reshape(o.shape[0] * 8, DOUT)


def _rows(i):
    return (i, 0)


def _whole(i):
    return (0, 0)


def _pk(shape=(RB, 128)):
    return pl.BlockSpec(shape, _rows)


def _w(shape):
    return pl.BlockSpec(shape, _whole)


def kernel(x, edge_index, gcn_W0, gcn_b0, gcn_W1, gcn_b1, gcn_W2, gcn_b2,
           red_W0, red_b0, red_W1, red_b1, gate_W0, gate_W1):
    x = x.astype(_f32)
    ei = edge_index.astype(jnp.int32)
    src2 = ei[0].reshape(ECH, CK)
    dst2 = ei[1].reshape(ECH, CK)
    xk = jnp.pad(x, ((0, NP - N), (0, 0))).reshape(RP, 8 * DIN)

    praw = pl.pallas_call(
        _prepa_body,
        grid=(GRID,),
        in_specs=[_pk((RB, 8 * DIN)), _w((DIN, DH)), _w((DIN, DH)),
                  _w((DIN, DH))],
        out_specs=_pk((RB, 384)),
        out_shape=jax.ShapeDtypeStruct((RP, 384), _f32),
        scratch_shapes=[pltpu.VMEM((8 * DIN, 384), _f32)],
    )(xk, gcn_W0, red_W0, red_W1)

    dg0, dg1 = _deg(dst2)

    v0p, r0p, r1p, dp = pl.pallas_call(
        _prepb_body,
        grid=(GRID,),
        in_specs=[_pk((RB, 384)), _w((1, DH)), _w((1, DH)), _pk(), _pk()],
        out_specs=(_pk(), _pk(), _pk(), _pk()),
        out_shape=tuple(jax.ShapeDtypeStruct((RP, 128), _f32) for _ in range(4)),
    )(praw, red_b0.reshape(1, DH), red_b1.reshape(1, DH),
      dg0.reshape(RP, 128), dg1.reshape(RP, 128))

    def mid(last, a0, a1, vp, h0p, b, gw, w1):
        return pl.pallas_call(
            functools.partial(_mid_body, last),
            grid=(GRID,),
            in_specs=[_pk(), _pk(), _pk(), _pk(), _pk(), _w((1, DH)),
                      _w((DH, 1)), _w((DH, 1)), _w((DH, DH))],
            out_specs=_pk(),
            out_shape=jax.ShapeDtypeStruct((RP, 128), _f32),
            scratch_shapes=[pltpu.VMEM((128, 8), _f32),
                            pltpu.VMEM((128, 8), _f32),
                            pltpu.VMEM((8, 128), _f32),
                            pltpu.VMEM((128, 128), _f32)],
        )(a0.reshape(RP, 128), a1.reshape(RP, 128), vp, dp, h0p,
          b.reshape(1, DH), gw[:DH], gw[DH:], w1)

    a0, a1 = _agg(v0p.reshape(NP, L), src2, dst2)
    v1p = mid(False, a0, a1, v0p, r0p, gcn_b0, gate_W0, gcn_W1)

    a0, a1 = _agg(v1p.reshape(NP, L), src2, dst2)
    v2p = mid(True, a0, a1, v1p, r1p, gcn_b1, gate_W1, gcn_W1)

    a0, a1 = _agg(v2p.reshape(NP, L), src2, dst2)
    out = pl.pallas_call(
        _final_body,
        grid=(GRID,),
        in_specs=[_pk(), _pk(), _pk(), _pk(), _w((DH, DOUT)), _w((1, DOUT))],
        out_specs=pl.BlockSpec((RB * 8, DOUT), _rows),
        out_shape=jax.ShapeDtypeStruct((N, DOUT), _f32),
        scratch_shapes=[pltpu.VMEM((128, 8 * DOUT), _f32)],
    )(a0.reshape(RP, 128), a1.reshape(RP, 128), v2p, dp,
      gcn_W2, gcn_b2.reshape(1, DOUT))

    return out


# speedup vs baseline: 80.8309x; 1.1342x over previous
import functools

import jax
import jax.numpy as jnp
from jax import lax
from jax.experimental import pallas as pl
from jax.experimental.pallas import tpu as pltpu
from jax.experimental.pallas import tpu_sc as plsc

N = 10000
E = 320000
DIN = 128
DH = 16
DOUT = 64

NC, NS, L = 2, 16, 16
NW = NC * NS
NP = 10240
RP = NP // 8
ROWS_S = NP // NS
CK = 125
ECH = E // CK
WCH = ECH // NW
GRID = 8
RB = RP // GRID

_f32 = jnp.float32
_mesh = plsc.VectorSubcoreMesh(core_axis_name="c", subcore_axis_name="s",
                               num_cores=NC, num_subcores=NS)
_sc_params = pltpu.CompilerParams(use_tc_tiling_on_sc=False)


def _zero_acc_slice(stage_v, acc_sh, sid):
    def zrow(i, c):
        stage_v[i, :] = jnp.zeros((L,), _f32)
        return c
    lax.fori_loop(0, ROWS_S, zrow, 0)
    pltpu.sync_copy(stage_v, acc_sh.at[pl.ds(sid * ROWS_S, ROWS_S)])


def _writeout(acc_sh, out0_hbm, out1_hbm, cid, sid):
    sl = pl.ds(sid * ROWS_S, ROWS_S)

    @pl.when(cid == 0)
    def _():
        pltpu.sync_copy(acc_sh.at[sl], out0_hbm.at[sl])

    @pl.when(cid == 1)
    def _():
        pltpu.sync_copy(acc_sh.at[sl], out1_hbm.at[sl])


@functools.partial(
    pl.kernel,
    out_type=(jax.ShapeDtypeStruct((NP, L), _f32),
              jax.ShapeDtypeStruct((NP, L), _f32)),
    mesh=_mesh,
    compiler_params=_sc_params,
    scratch_types=[
        pltpu.VMEM((WCH, CK), jnp.int32),
        pltpu.VMEM((CK, L), _f32),
        pltpu.VMEM((ROWS_S, L), _f32),
        pltpu.VMEM_SHARED((NP, L), _f32),
    ],
)
def _deg(edge_hbm, out0_hbm, out1_hbm, dst_v, ones_v, stage_v, acc_sh):
    cid = lax.axis_index("c")
    sid = lax.axis_index("s")
    wid = cid * NS + sid

    _zero_acc_slice(stage_v, acc_sh, sid)

    def orow(i, c):
        ones_v[i, :] = jnp.ones((L,), _f32)
        return c
    lax.fori_loop(0, CK, orow, 0)
    pltpu.sync_copy(edge_hbm.at[1, pl.ds(wid * WCH, WCH)], dst_v)
    plsc.subcore_barrier()

    def step(j, c):
        pltpu.sync_copy(ones_v, acc_sh.at[dst_v.at[j]], add=True)
        return c
    lax.fori_loop(0, WCH, step, 0)

    plsc.subcore_barrier()
    _writeout(acc_sh, out0_hbm, out1_hbm, cid, sid)


@functools.partial(
    pl.kernel,
    out_type=(jax.ShapeDtypeStruct((NP, L), _f32),
              jax.ShapeDtypeStruct((NP, L), _f32)),
    mesh=_mesh,
    compiler_params=_sc_params,
    scratch_types=[
        pltpu.VMEM((WCH, CK), jnp.int32),
        pltpu.VMEM((WCH, CK), jnp.int32),
        pltpu.VMEM((4, CK, L), _f32),
        pltpu.VMEM((ROWS_S, L), _f32),
        pltpu.VMEM_SHARED((NP, L), _f32),
        pltpu.VMEM_SHARED((NP, L), _f32),
        pltpu.SemaphoreType.DMA,
        pltpu.SemaphoreType.DMA,
    ],
)
def _agg(tab_hbm, edge_hbm, out0_hbm, out1_hbm,
         src_v, dst_v, rows_v, stage_v, acc_sh, tab_sh, gsem, ssem):
    cid = lax.axis_index("c")
    sid = lax.axis_index("s")
    wid = cid * NS + sid
    sl = pl.ds(sid * ROWS_S, ROWS_S)

    _zero_acc_slice(stage_v, acc_sh, sid)
    pltpu.sync_copy(tab_hbm.at[sl], tab_sh.at[sl])
    pltpu.sync_copy(edge_hbm.at[0, pl.ds(wid * WCH, WCH)], src_v)
    pltpu.sync_copy(edge_hbm.at[1, pl.ds(wid * WCH, WCH)], dst_v)
    plsc.subcore_barrier()

    pltpu.async_copy(tab_sh.at[src_v.at[0]], rows_v.at[0], gsem)
    pltpu.async_copy(tab_sh.at[src_v.at[1]], rows_v.at[1], gsem)

    def step(j, c):
        b = lax.rem(j, 4)
        pltpu.make_async_copy(tab_sh.at[src_v.at[j]], rows_v.at[b], gsem).wait()
        pltpu.async_copy(rows_v.at[b], acc_sh.at[dst_v.at[j]], ssem, add=True)

        @pl.when(j >= 2)
        def _():
            pltpu.make_async_copy(
                rows_v.at[lax.rem(j + 2, 4)],
                acc_sh.at[dst_v.at[j - 2]], ssem).wait()

        @pl.when(j + 2 < WCH)
        def _():
            pltpu.async_copy(tab_sh.at[src_v.at[j + 2]],
                             rows_v.at[lax.rem(j + 2, 4)], gsem)
        return c
    lax.fori_loop(0, WCH, step, 0)

    pltpu.make_async_copy(rows_v.at[lax.rem(WCH - 2, 4)],
                          acc_sh.at[dst_v.at[WCH - 2]], ssem).wait()
    pltpu.make_async_copy(rows_v.at[lax.rem(WCH - 1, 4)],
                          acc_sh.at[dst_v.at[WCH - 1]], ssem).wait()

    plsc.subcore_barrier()
    _writeout(acc_sh, out0_hbm, out1_hbm, cid, sid)


def _prepa_body(xk_ref, w0_ref, wr0_ref, wr1_ref, p_ref, kall_v):
    @pl.when(pl.program_id(0) == 0)
    def _():
        kall_v[...] = jnp.zeros((8 * DIN, 384), _f32)
        for a in range(8):
            r = pl.ds(a * DIN, DIN)
            kall_v[r, pl.ds(a * DH, DH)] = w0_ref[...]
            kall_v[r, pl.ds(128 + a * DH, DH)] = wr0_ref[...]
            kall_v[r, pl.ds(256 + a * DH, DH)] = wr1_ref[...]

    p_ref[...] = jnp.dot(xk_ref[...], kall_v[...], preferred_element_type=_f32)


def _prepb_body(p_ref, b0_ref, b1_ref, dg0_ref, dg1_ref,
                v0_ref, r0_ref, r1_ref, d_ref):
    deg = dg0_ref[...] + dg1_ref[...] + 1.0
    dinv = lax.rsqrt(deg)
    d_ref[...] = dinv
    p = p_ref[...]
    tb0 = jnp.tile(b0_ref[...], (1, 8))
    tb1 = jnp.tile(b1_ref[...], (1, 8))
    v0_ref[...] = dinv * p[:, :128]
    r0_ref[...] = p[:, 128:256] + tb0
    r1_ref[...] = p[:, 256:384] + tb1


def _mid_body(last, a0_ref, a1_ref, v_ref, d_ref, h0_ref, b_ref, gwa_ref,
              gwb_ref, w1_ref, vout_ref, kga_v, kgb_v, kb8_v, kw1_v):
    @pl.when(pl.program_id(0) == 0)
    def _():
        kga_v[...] = jnp.zeros((128, 8), _f32)
        kgb_v[...] = jnp.zeros((128, 8), _f32)
        kb8_v[...] = jnp.zeros((8, 128), _f32)
        kw1_v[...] = jnp.zeros((128, 128), _f32)
        for a in range(8):
            r = pl.ds(a * DH, DH)
            kga_v[r, pl.ds(a, 1)] = gwa_ref[...]
            kgb_v[r, pl.ds(a, 1)] = gwb_ref[...]
            kb8_v[pl.ds(a, 1), r] = jnp.ones((1, DH), _f32)
            kw1_v[r, r] = w1_ref[...]

    d = d_ref[...]
    h1 = jnp.maximum(
        d * (a0_ref[...] + a1_ref[...] + v_ref[...])
        + jnp.tile(b_ref[...], (1, 8)), 0.0)
    h0 = h0_ref[...]
    if last:
        gwa, gwb = gwa_ref[...], gwb_ref[...]
        scale = lax.rsqrt(jnp.sum(gwa * gwa) + jnp.sum(gwb * gwb))
    else:
        scale = 1.0
    zz = (jnp.dot(h1, kga_v[...], preferred_element_type=_f32)
          + jnp.dot(h0, kgb_v[...], preferred_element_type=_f32)) * scale
    g = jax.nn.sigmoid(zz)
    gp = jnp.dot(g, kb8_v[...], preferred_element_type=_f32)
    h = gp * h1 + (1.0 - gp) * h0
    if last:
        vout_ref[...] = d * h
    else:
        vout_ref[...] = d * jnp.dot(h, kw1_v[...], preferred_element_type=_f32)


def _final_body(a0_ref, a1_ref, v_ref, d_ref, w2_ref, b2_ref, out_ref, kw2_v):
    @pl.when(pl.program_id(0) == 0)
    def _():
        kw2_v[...] = jnp.zeros((128, 8 * DOUT), _f32)
        for a in range(8):
            kw2_v[pl.ds(a * DH, DH), pl.ds(a * DOUT, DOUT)] = w2_ref[...]

    a2 = d_ref[...] * (a0_ref[...] + a1_ref[...] + v_ref[...])
    o = (jnp.dot(a2, kw2_v[...], preferred_element_type=_f32)
         + jnp.tile(b2_ref[...], (1, 8)))
    o3 = o.reshape(o.shape[0], 8, DOUT)
    m = jnp.max(o3, axis=2, keepdims=True)
    e = o3 - m
    ls = jnp.log(jnp.sum(jnp.exp(e), axis=2, keepdims=True))
    out_ref[...] = (e - ls).reshape(o.shape[0] * 8, DOUT)


def _rows(i):
    return (i, 0)


def _whole(i):
    return (0, 0)


def _pk(shape=(RB, 128)):
    return pl.BlockSpec(shape, _rows)


def _w(shape):
    return pl.BlockSpec(shape, _whole)


def kernel(x, edge_index, gcn_W0, gcn_b0, gcn_W1, gcn_b1, gcn_W2, gcn_b2,
           red_W0, red_b0, red_W1, red_b1, gate_W0, gate_W1):
    x = x.astype(_f32)
    e3 = edge_index.astype(jnp.int32).reshape(2, ECH, CK)
    xk = jnp.pad(x, ((0, NP - N), (0, 0))).reshape(RP, 8 * DIN)

    praw = pl.pallas_call(
        _prepa_body,
        grid=(GRID,),
        in_specs=[_pk((RB, 8 * DIN)), _w((DIN, DH)), _w((DIN, DH)),
                  _w((DIN, DH))],
        out_specs=_pk((RB, 384)),
        out_shape=jax.ShapeDtypeStruct((RP, 384), _f32),
        scratch_shapes=[pltpu.VMEM((8 * DIN, 384), _f32)],
    )(xk, gcn_W0, red_W0, red_W1)

    dg0, dg1 = _deg(e3)

    v0p, r0p, r1p, dp = pl.pallas_call(
        _prepb_body,
        grid=(GRID,),
        in_specs=[_pk((RB, 384)), _w((1, DH)), _w((1, DH)), _pk(), _pk()],
        out_specs=(_pk(), _pk(), _pk(), _pk()),
        out_shape=tuple(jax.ShapeDtypeStruct((RP, 128), _f32) for _ in range(4)),
    )(praw, red_b0.reshape(1, DH), red_b1.reshape(1, DH),
      dg0.reshape(RP, 128), dg1.reshape(RP, 128))

    def mid(last, a0, a1, vp, h0p, b, gw, w1):
        return pl.pallas_call(
            functools.partial(_mid_body, last),
            grid=(GRID,),
            in_specs=[_pk(), _pk(), _pk(), _pk(), _pk(), _w((1, DH)),
                      _w((DH, 1)), _w((DH, 1)), _w((DH, DH))],
            out_specs=_pk(),
            out_shape=jax.ShapeDtypeStruct((RP, 128), _f32),
            scratch_shapes=[pltpu.VMEM((128, 8), _f32),
                            pltpu.VMEM((128, 8), _f32),
                            pltpu.VMEM((8, 128), _f32),
                            pltpu.VMEM((128, 128), _f32)],
        )(a0.reshape(RP, 128), a1.reshape(RP, 128), vp, dp, h0p,
          b.reshape(1, DH), gw[:DH], gw[DH:], w1)

    a0, a1 = _agg(v0p.reshape(NP, L), e3)
    v1p = mid(False, a0, a1, v0p, r0p, gcn_b0, gate_W0, gcn_W1)

    a0, a1 = _agg(v1p.reshape(NP, L), e3)
    v2p = mid(True, a0, a1, v1p, r1p, gcn_b1, gate_W1, gcn_W1)

    a0, a1 = _agg(v2p.reshape(NP, L), e3)
    out = pl.pallas_call(
        _final_body,
        grid=(GRID,),
        in_specs=[_pk(), _pk(), _pk(), _pk(), _w((DH, DOUT)), _w((1, DOUT))],
        out_specs=pl.BlockSpec((RB * 8, DOUT), _rows),
        out_shape=jax.ShapeDtypeStruct((N, DOUT), _f32),
        scratch_shapes=[pltpu.VMEM((128, 8 * DOUT), _f32)],
    )(a0.reshape(RP, 128), a1.reshape(RP, 128), v2p, dp,
      gcn_W2, gcn_b2.reshape(1, DOUT))

    return out
